# baseline (speedup 1.0000x reference)
import functools

import jax
import jax.numpy as jnp
from jax import lax
from jax.experimental import pallas as pl
from jax.experimental.pallas import tpu as pltpu
from jax.experimental.pallas import tpu_sc as plsc

NUM_ENT = 10000
NUM_REL = 100
D = 128
N_EDGES = 320000
BATCH = 1024
EPS = 1e-5

NC, NS, LANES = 2, 16, 16
CK = 128
HALF = N_EDGES // 2
NCHUNK = 81
EPT = NCHUNK * CK
NBUF = 3
NREL = 2 * NUM_REL
ENT_PAD = 10240
RPT = ENT_PAD // NS


DH = D // 2
VPG = DH // LANES


def _sc_accum_body(x0_hbm, x1_hbm, rel0_hbm, rel1_hbm, edges_hbm, nrep_hbm,
                   out_hbm,
                   accum, idx_all, xrows0, xrows1, xrows2,
                   rrows0, rrows1, rrows2, nrep0, nrep1, nrep2,
                   esem, gsem0, gsem1, gsem2, rsem0, rsem1, rsem2,
                   nsem0, nsem1, nsem2, ssem0, ssem1, ssem2):
    c = lax.axis_index("c")
    s = lax.axis_index("s")
    xrows = (xrows0, xrows1, xrows2)
    rrows = (rrows0, rrows1, rrows2)
    nrep = (nrep0, nrep1, nrep2)
    gsem = (gsem0, gsem1, gsem2)
    rsem = (rsem0, rsem1, rsem2)
    nsem = (nsem0, nsem1, nsem2)
    ssem = (ssem0, ssem1, ssem2)

    cpe = pltpu.async_copy(edges_hbm.at[c, s], idx_all, esem)

    def _zero_buf():
        def _zero_row(j, carry):
            for v in range(VPG):
                xrows0[j, pl.ds(v * LANES, LANES)] = jnp.zeros((LANES,),
                                                               jnp.float32)
            return carry

        lax.fori_loop(0, CK, _zero_row, 0)

    def _zero_accum():
        for k in range(RPT // CK):
            pltpu.sync_copy(xrows0, accum.at[pl.ds(s * RPT + k * CK, CK)])

    _zero_buf()
    _zero_accum()
    cpe.wait()
    plsc.subcore_barrier()

    for p in range(2):
        xp_hbm = (x0_hbm, x1_hbm)[p]
        rp_hbm = (rel0_hbm, rel1_hbm)[p]

        def _start_fetch(g, b):
            pltpu.async_copy(xp_hbm.at[idx_all.at[g, 0]], xrows[b], gsem[b])
            pltpu.async_copy(rp_hbm.at[idx_all.at[g, 1]], rrows[b], rsem[b])
            pltpu.async_copy(nrep_hbm.at[c, s, g], nrep[b], nsem[b])

        def _drain_fetch(g, b):
            pltpu.make_async_copy(xp_hbm.at[idx_all.at[g, 0]], xrows[b],
                                  gsem[b]).wait()
            pltpu.make_async_copy(rp_hbm.at[idx_all.at[g, 1]], rrows[b],
                                  rsem[b]).wait()
            pltpu.make_async_copy(nrep_hbm.at[c, s, g], nrep[b],
                                  nsem[b]).wait()

        def _drain_scatter(b):
            pltpu.make_async_copy(xrows[b], accum.at[idx_all.at[0, 2]],
                                  ssem[b]).wait()

        _start_fetch(0, 0)
        _start_fetch(1, 1)

        def _compute(g, b):
            xb = xrows[b]
            rb = rrows[b]
            nb = nrep[b]

            def _group(g2, carry):
                for e in range(LANES):
                    j = g2 * LANES + e
                    nvv = nb[j]
                    for v in range(VPG):
                        sl = pl.ds(v * LANES, LANES)
                        xb[j, sl] = xb[j, sl] * rb[j, sl] * nvv
                return carry

            lax.fori_loop(0, CK // LANES, _group, 0)

        def _outer(io, carry):
            for b in range(NBUF):
                g = io * NBUF + b
                _drain_fetch(g, b)
                _compute(g, b)
                pltpu.async_copy(xrows[b], accum.at[idx_all.at[g, 2]],
                                 ssem[b], add=True)
                bp = (b + 2) % NBUF

                @pl.when(jnp.logical_and(g >= 1, g + 2 < NCHUNK))
                def _pref():
                    _drain_scatter(bp)
                    _start_fetch(g + 2, bp)

                @pl.when(g == 0)
                def _pref0():
                    _start_fetch(2, bp)

            return carry

        lax.fori_loop(0, NCHUNK // NBUF, _outer, 0)
        for b in range(NBUF):
            _drain_scatter(b)
        plsc.subcore_barrier()

        for k in range(RPT // CK):
            r0 = s * RPT + k * CK
            pltpu.sync_copy(accum.at[pl.ds(r0, CK)],
                            out_hbm.at[c, p, pl.ds(r0, CK)])
        if p == 0:
            _zero_buf()
            _zero_accum()
            plsc.subcore_barrier()


def _sc_accum(x, rel_emb, edges, nrep):
    mesh = plsc.VectorSubcoreMesh(core_axis_name="c", subcore_axis_name="s")
    fn = pl.kernel(
        _sc_accum_body,
        out_type=jax.ShapeDtypeStruct((NC, 2, ENT_PAD, DH), jnp.float32),
        mesh=mesh,
        scratch_types=[
            pltpu.VMEM_SHARED((ENT_PAD, DH), jnp.float32),
            pltpu.VMEM((NCHUNK, 3, CK), jnp.int32),
            pltpu.VMEM((CK, DH), jnp.float32),
            pltpu.VMEM((CK, DH), jnp.float32),
            pltpu.VMEM((CK, DH), jnp.float32),
            pltpu.VMEM((CK, DH), jnp.float32),
            pltpu.VMEM((CK, DH), jnp.float32),
            pltpu.VMEM((CK, DH), jnp.float32),
            pltpu.VMEM((CK, LANES), jnp.float32),
            pltpu.VMEM((CK, LANES), jnp.float32),
            pltpu.VMEM((CK, LANES), jnp.float32),
        ] + [pltpu.SemaphoreType.DMA] * 13,
        compiler_params=pltpu.CompilerParams(use_tc_tiling_on_sc=False),
    )
    out = fn(x[:, :DH], x[:, DH:], rel_emb[:, :DH], rel_emb[:, DH:],
             edges, nrep)
    return jnp.concatenate([out[:, 0], out[:, 1]], axis=2)


def _tc_layer_body(ain, aout, x, rel, in_w, out_w, loop_w, loop_rel, bias,
                   gamma, beta, w_rel, xo, ro):
    f32 = jnp.float32
    agg = jnp.dot(ain[...], in_w[...], preferred_element_type=f32)
    agg = agg + jnp.dot(aout[...], out_w[...], preferred_element_type=f32)
    xl = x[...] * loop_rel[...]
    agg = agg + jnp.dot(xl, loop_w[...], preferred_element_type=f32) / 3.0
    xn = agg + bias[...]
    mu = jnp.mean(xn, axis=0, keepdims=True)
    var = jnp.mean((xn - mu) * (xn - mu), axis=0, keepdims=True)
    xn = (xn - mu) * lax.rsqrt(var + EPS) * gamma[...] + beta[...]
    xo[...] = jnp.tanh(xn)
    ro[...] = jnp.dot(rel[...], w_rel[...], preferred_element_type=f32)


def _tc_layer(ain, aout, x, rel, in_w, out_w, loop_w, loop_rel, bias, gamma,
              beta, w_rel):
    nrel = rel.shape[0]
    return pl.pallas_call(
        _tc_layer_body,
        out_shape=(
            jax.ShapeDtypeStruct((NUM_ENT, D), jnp.float32),
            jax.ShapeDtypeStruct((nrel, D), jnp.float32),
        ),
    )(ain, aout, x, rel, in_w, out_w, loop_w, loop_rel.reshape(1, D),
      bias.reshape(1, D), gamma.reshape(1, D), beta.reshape(1, D), w_rel)


def _tc_relmm_body(a, b, o):
    o[...] = jnp.dot(a[...], b[...], preferred_element_type=jnp.float32)


def _tc_relmm(a, b):
    return pl.pallas_call(
        _tc_relmm_body,
        out_shape=jax.ShapeDtypeStruct((a.shape[0], b.shape[1]), jnp.float32),
    )(a, b)


BPW = BATCH // (NC * NS)


def _sc_gather_body(x_hbm, r_hbm, subj_hbm, rel_hbm, obj_hbm,
                    so_hbm, ro_hbm, oo_hbm, idx_v, rows_v, sem):
    c = lax.axis_index("c")
    s = lax.axis_index("s")
    base = (s * NC + c) * BPW
    for ih, table, oh in ((subj_hbm, x_hbm, so_hbm),
                          (rel_hbm, r_hbm, ro_hbm),
                          (obj_hbm, x_hbm, oo_hbm)):
        pltpu.sync_copy(ih.at[pl.ds(base, BPW)], idx_v)
        pltpu.async_copy(table.at[idx_v], rows_v, sem).wait()
        pltpu.sync_copy(rows_v, oh.at[pl.ds(base, BPW)])


def _sc_gather(x, r, subj, rel, obj):
    mesh = plsc.VectorSubcoreMesh(core_axis_name="c", subcore_axis_name="s")
    fn = pl.kernel(
        _sc_gather_body,
        out_type=(
            jax.ShapeDtypeStruct((BATCH, D), jnp.float32),
            jax.ShapeDtypeStruct((BATCH, D), jnp.float32),
            jax.ShapeDtypeStruct((BATCH, D), jnp.float32),
        ),
        mesh=mesh,
        scratch_types=[
            pltpu.VMEM((BPW,), jnp.int32),
            pltpu.VMEM((BPW, D), jnp.float32),
            pltpu.SemaphoreType.DMA,
        ],
    )
    return fn(x, r, subj, rel, obj)


def _prep_edges(edge_index, edge_type, edge_norm):
    pad = NS * EPT - HALF

    def half(a, fill):
        a0 = jnp.concatenate([a[:HALF], jnp.full((pad,), fill, a.dtype)])
        a1 = jnp.concatenate([a[HALF:], jnp.full((pad,), fill, a.dtype)])
        return jnp.stack([a0, a1]).reshape(NC, NS, NCHUNK, CK)

    src = half(edge_index[0].astype(jnp.int32), 0)
    typ = half(edge_type.astype(jnp.int32), 0)
    dst = half(edge_index[1].astype(jnp.int32), 0)
    nrm = half(edge_norm.astype(jnp.float32), 0.0)
    nrep = jnp.broadcast_to(nrm[..., None],
                            (NC, NS, NCHUNK, CK, LANES))
    return jnp.stack([src, typ, dst], axis=3), nrep


def kernel(edge_index, edge_type, edge_norm, subj, rel, obj, init_embed,
           init_rel, rel_wt, in_w1, out_w1, loop_w1, w_rel1, loop_rel1, bias1,
           gamma1, beta1, in_w2, out_w2, loop_w2, w_rel2, loop_rel2, bias2,
           gamma2, beta2):
    edges, nrm = _prep_edges(edge_index, edge_type, edge_norm)

    rel1 = _tc_relmm(rel_wt, init_rel)

    acc = _sc_accum(init_embed, rel1, edges, nrm)
    x1, rel2 = _tc_layer(acc[0, :NUM_ENT], acc[1, :NUM_ENT], init_embed, rel1,
                         in_w1, out_w1, loop_w1, loop_rel1, bias1, gamma1,
                         beta1, w_rel1)

    acc = _sc_accum(x1, rel2, edges, nrm)
    x2, rel3 = _tc_layer(acc[0, :NUM_ENT], acc[1, :NUM_ENT], x1, rel2,
                         in_w2, out_w2, loop_w2, loop_rel2, bias2, gamma2,
                         beta2, w_rel2)

    return _sc_gather(x2, rel3, subj.astype(jnp.int32), rel.astype(jnp.int32),
                      obj.astype(jnp.int32))


# speedup vs baseline: 1.1057x; 1.1057x over previous
import functools

import jax
import jax.numpy as jnp
from jax import lax
from jax.experimental import pallas as pl
from jax.experimental.pallas import tpu as pltpu
from jax.experimental.pallas import tpu_sc as plsc

NUM_ENT = 10000
NUM_REL = 100
D = 128
N_EDGES = 320000
BATCH = 1024
EPS = 1e-5

NC, NS, LANES = 2, 16, 16
CK = 128
HALF = N_EDGES // 2
NCHUNK = 81
EPT = NCHUNK * CK
NBUF = 3
NREL = 2 * NUM_REL
ENT_PAD = 10240
RPT = ENT_PAD // NS


DH = D // 2
VPG = DH // LANES


def _sc_accum_body(x0_hbm, x1_hbm, rel_hbm, edges_hbm, nrm_hbm, out_hbm,
                   accum, rel_v, idx_all, nrm_all, xrows0, xrows1, xrows2,
                   esem, rsem, nsem, gsem0, gsem1, gsem2,
                   ssem0, ssem1, ssem2):
    c = lax.axis_index("c")
    s = lax.axis_index("s")
    xrows = (xrows0, xrows1, xrows2)
    gsem = (gsem0, gsem1, gsem2)
    ssem = (ssem0, ssem1, ssem2)

    cpe = pltpu.async_copy(edges_hbm.at[c, s], idx_all, esem)
    cpn = pltpu.async_copy(nrm_hbm.at[c, s], nrm_all, nsem)

    def _zero_buf():
        def _zero_row(j, carry):
            for v in range(VPG):
                xrows0[j, pl.ds(v * LANES, LANES)] = jnp.zeros((LANES,),
                                                               jnp.float32)
            return carry

        lax.fori_loop(0, CK, _zero_row, 0)

    def _zero_accum():
        for k in range(RPT // CK):
            pltpu.sync_copy(xrows0, accum.at[pl.ds(s * RPT + k * CK, CK)])

    _zero_buf()
    _zero_accum()
    cpe.wait()
    cpn.wait()
    plsc.subcore_barrier()

    for p in range(2):
        xp_hbm = (x0_hbm, x1_hbm)[p]
        pltpu.sync_copy(rel_hbm.at[p], rel_v)

        def _start_gather(g, b):
            pltpu.async_copy(xp_hbm.at[idx_all.at[g, 0]], xrows[b], gsem[b])

        def _drain_gather(g, b):
            pltpu.make_async_copy(xp_hbm.at[idx_all.at[g, 0]], xrows[b],
                                  gsem[b]).wait()

        def _drain_scatter(b):
            pltpu.make_async_copy(xrows[b], accum.at[idx_all.at[0, 2]],
                                  ssem[b]).wait()

        _start_gather(0, 0)
        _start_gather(1, 1)

        def _compute(g, b):
            xb = xrows[b]

            def _group(g2, carry):
                ty16 = idx_all[g, 1, pl.ds(g2 * LANES, LANES)]
                nv16 = nrm_all[g, pl.ds(g2 * LANES, LANES)]
                for e in range(LANES):
                    j = g2 * LANES + e
                    nv = nv16[e]
                    ty = ty16[e]
                    for v in range(VPG):
                        sl = pl.ds(v * LANES, LANES)
                        xb[j, sl] = xb[j, sl] * rel_v[ty, sl] * nv
                return carry

            lax.fori_loop(0, CK // LANES, _group, 0)

        def _outer(io, carry):
            for b in range(NBUF):
                g = io * NBUF + b
                _drain_gather(g, b)
                _compute(g, b)
                pltpu.async_copy(xrows[b], accum.at[idx_all.at[g, 2]],
                                 ssem[b], add=True)
                bp = (b + 2) % NBUF

                @pl.when(jnp.logical_and(g >= 1, g + 2 < NCHUNK))
                def _pref():
                    _drain_scatter(bp)
                    _start_gather(g + 2, bp)

                @pl.when(g == 0)
                def _pref0():
                    _start_gather(2, bp)

            return carry

        lax.fori_loop(0, NCHUNK // NBUF, _outer, 0)
        for b in range(NBUF):
            _drain_scatter(b)
        plsc.subcore_barrier()

        for k in range(RPT // CK):
            r0 = s * RPT + k * CK
            pltpu.sync_copy(accum.at[pl.ds(r0, CK)],
                            out_hbm.at[c, p, pl.ds(r0, CK)])
        if p == 0:
            _zero_buf()
            _zero_accum()
            plsc.subcore_barrier()


def _sc_accum(x0, x1, rel_emb, edges, nrm):
    mesh = plsc.VectorSubcoreMesh(core_axis_name="c", subcore_axis_name="s")
    fn = pl.kernel(
        _sc_accum_body,
        out_type=jax.ShapeDtypeStruct((NC, 2, ENT_PAD, DH), jnp.float32),
        mesh=mesh,
        scratch_types=[
            pltpu.VMEM_SHARED((ENT_PAD, DH), jnp.float32),
            pltpu.VMEM((NREL, DH), jnp.float32),
            pltpu.VMEM((NCHUNK, 3, CK), jnp.int32),
            pltpu.VMEM((NCHUNK, CK), jnp.float32),
            pltpu.VMEM((CK, DH), jnp.float32),
            pltpu.VMEM((CK, DH), jnp.float32),
            pltpu.VMEM((CK, DH), jnp.float32),
        ] + [pltpu.SemaphoreType.DMA] * 9,
        compiler_params=pltpu.CompilerParams(use_tc_tiling_on_sc=False),
    )
    relh = rel_emb.reshape(NREL, 2, DH).transpose(1, 0, 2)
    return fn(x0, x1, relh, edges, nrm)


def _tc_layer_body(acc, x, rel, in_w, out_w, loop_w, loop_rel, bias,
                   gamma, beta, w_rel, xo, x0o, x1o, ro):
    f32 = jnp.float32
    agg = jnp.dot(acc[0, 0, :NUM_ENT, :], in_w[:DH, :],
                  preferred_element_type=f32)
    agg = agg + jnp.dot(acc[0, 1, :NUM_ENT, :], in_w[DH:, :],
                        preferred_element_type=f32)
    agg = agg + jnp.dot(acc[1, 0, :NUM_ENT, :], out_w[:DH, :],
                        preferred_element_type=f32)
    agg = agg + jnp.dot(acc[1, 1, :NUM_ENT, :], out_w[DH:, :],
                        preferred_element_type=f32)
    xl = x[...] * loop_rel[...]
    agg = agg + jnp.dot(xl, loop_w[...], preferred_element_type=f32) / 3.0
    xn = agg + bias[...]
    mu = jnp.mean(xn, axis=0, keepdims=True)
    var = jnp.mean((xn - mu) * (xn - mu), axis=0, keepdims=True)
    xn = (xn - mu) * lax.rsqrt(var + EPS) * gamma[...] + beta[...]
    xt = jnp.tanh(xn)
    xo[...] = xt
    x0o[...] = xt[:, :DH]
    x1o[...] = xt[:, DH:]
    ro[...] = jnp.dot(rel[...], w_rel[...], preferred_element_type=f32)


def _tc_layer(acc, x, rel, in_w, out_w, loop_w, loop_rel, bias, gamma,
              beta, w_rel):
    nrel = rel.shape[0]
    return pl.pallas_call(
        _tc_layer_body,
        out_shape=(
            jax.ShapeDtypeStruct((NUM_ENT, D), jnp.float32),
            jax.ShapeDtypeStruct((NUM_ENT, DH), jnp.float32),
            jax.ShapeDtypeStruct((NUM_ENT, DH), jnp.float32),
            jax.ShapeDtypeStruct((nrel, D), jnp.float32),
        ),
    )(acc, x, rel, in_w, out_w, loop_w, loop_rel.reshape(1, D),
      bias.reshape(1, D), gamma.reshape(1, D), beta.reshape(1, D), w_rel)


def _tc_relmm_body(a, b, o):
    o[...] = jnp.dot(a[...], b[...], preferred_element_type=jnp.float32)


def _tc_relmm(a, b):
    return pl.pallas_call(
        _tc_relmm_body,
        out_shape=jax.ShapeDtypeStruct((a.shape[0], b.shape[1]), jnp.float32),
    )(a, b)


BPW = BATCH // (NC * NS)


def _sc_gather_body(x_hbm, r_hbm, subj_hbm, rel_hbm, obj_hbm,
                    so_hbm, ro_hbm, oo_hbm, idx_v, rows_v, sem):
    c = lax.axis_index("c")
    s = lax.axis_index("s")
    base = (s * NC + c) * BPW
    for ih, table, oh in ((subj_hbm, x_hbm, so_hbm),
                          (rel_hbm, r_hbm, ro_hbm),
                          (obj_hbm, x_hbm, oo_hbm)):
        pltpu.sync_copy(ih.at[pl.ds(base, BPW)], idx_v)
        pltpu.async_copy(table.at[idx_v], rows_v, sem).wait()
        pltpu.sync_copy(rows_v, oh.at[pl.ds(base, BPW)])


def _sc_gather(x, r, subj, rel, obj):
    mesh = plsc.VectorSubcoreMesh(core_axis_name="c", subcore_axis_name="s")
    fn = pl.kernel(
        _sc_gather_body,
        out_type=(
            jax.ShapeDtypeStruct((BATCH, D), jnp.float32),
            jax.ShapeDtypeStruct((BATCH, D), jnp.float32),
            jax.ShapeDtypeStruct((BATCH, D), jnp.float32),
        ),
        mesh=mesh,
        scratch_types=[
            pltpu.VMEM((BPW,), jnp.int32),
            pltpu.VMEM((BPW, D), jnp.float32),
            pltpu.SemaphoreType.DMA,
        ],
    )
    return fn(x, r, subj, rel, obj)


def _prep_edges(edge_index, edge_type, edge_norm):
    pad = NS * EPT - HALF

    def half(a, fill):
        a0 = jnp.concatenate([a[:HALF], jnp.full((pad,), fill, a.dtype)])
        a1 = jnp.concatenate([a[HALF:], jnp.full((pad,), fill, a.dtype)])
        return jnp.stack([a0, a1]).reshape(NC, NS, NCHUNK, CK)

    src = half(edge_index[0].astype(jnp.int32), 0)
    typ = half(edge_type.astype(jnp.int32), 0)
    dst = half(edge_index[1].astype(jnp.int32), 0)
    nrm = half(edge_norm.astype(jnp.float32), 0.0)
    return jnp.stack([src, typ, dst], axis=3), nrm


def kernel(edge_index, edge_type, edge_norm, subj, rel, obj, init_embed,
           init_rel, rel_wt, in_w1, out_w1, loop_w1, w_rel1, loop_rel1, bias1,
           gamma1, beta1, in_w2, out_w2, loop_w2, w_rel2, loop_rel2, bias2,
           gamma2, beta2):
    edges, nrm = _prep_edges(edge_index, edge_type, edge_norm)

    rel1 = _tc_relmm(rel_wt, init_rel)

    acc = _sc_accum(init_embed[:, :DH], init_embed[:, DH:], rel1, edges, nrm)
    x1, x10, x11, rel2 = _tc_layer(acc, init_embed, rel1,
                                   in_w1, out_w1, loop_w1, loop_rel1, bias1,
                                   gamma1, beta1, w_rel1)

    acc = _sc_accum(x10, x11, rel2, edges, nrm)
    x2, _, _, rel3 = _tc_layer(acc, x1, rel2,
                               in_w2, out_w2, loop_w2, loop_rel2, bias2,
                               gamma2, beta2, w_rel2)

    return _sc_gather(x2, rel3, subj.astype(jnp.int32), rel.astype(jnp.int32),
                      obj.astype(jnp.int32))


# speedup vs baseline: 1.1120x; 1.0057x over previous
import functools

import jax
import jax.numpy as jnp
from jax import lax
from jax.experimental import pallas as pl
from jax.experimental.pallas import tpu as pltpu
from jax.experimental.pallas import tpu_sc as plsc

NUM_ENT = 10000
NUM_REL = 100
D = 128
N_EDGES = 320000
BATCH = 1024
EPS = 1e-5

NC, NS, LANES = 2, 16, 16
CK = 128
HALF = N_EDGES // 2
NCHUNK = 81
EPT = NCHUNK * CK
NBUF = 3
NREL = 2 * NUM_REL
ENT_PAD = 10240
RPT = ENT_PAD // NS


DH = D // 2
VPG = DH // LANES


def _sc_accum_body(x0_hbm, x1_hbm, rel_hbm, edges_hbm, nrm_hbm, out_hbm,
                   accum, rel_v, idx_all, nrm_all, xrows0, xrows1, xrows2,
                   esem, rsem, nsem, gsem0, gsem1, gsem2,
                   ssem0, ssem1, ssem2):
    c = lax.axis_index("c")
    s = lax.axis_index("s")
    xrows = (xrows0, xrows1, xrows2)
    gsem = (gsem0, gsem1, gsem2)
    ssem = (ssem0, ssem1, ssem2)

    cpe = pltpu.async_copy(edges_hbm.at[c, s], idx_all, esem)
    cpn = pltpu.async_copy(nrm_hbm.at[c, s], nrm_all, nsem)

    def _zero_buf():
        def _zero_row(j, carry):
            for v in range(VPG):
                xrows2[j, pl.ds(v * LANES, LANES)] = jnp.zeros((LANES,),
                                                               jnp.float32)
            return carry

        lax.fori_loop(0, CK, _zero_row, 0)

    def _zero_accum():
        for k in range(RPT // CK):
            pltpu.async_copy(xrows2, accum.at[pl.ds(s * RPT + k * CK, CK)],
                             rsem)
        for k in range(RPT // CK):
            pltpu.make_async_copy(
                xrows2, accum.at[pl.ds(s * RPT, CK)], rsem).wait()

    def _writeout(p):
        for k in range(RPT // CK):
            r0 = s * RPT + k * CK
            pltpu.async_copy(accum.at[pl.ds(r0, CK)],
                             out_hbm.at[c, p, pl.ds(r0, CK)], rsem)
        for k in range(RPT // CK):
            pltpu.make_async_copy(accum.at[pl.ds(s * RPT, CK)],
                                  out_hbm.at[c, p, pl.ds(s * RPT, CK)],
                                  rsem).wait()

    def _start_gather(xp_hbm, g, b):
        pltpu.async_copy(xp_hbm.at[idx_all.at[g, 0]], xrows[b], gsem[b])

    def _drain_gather(xp_hbm, g, b):
        pltpu.make_async_copy(xp_hbm.at[idx_all.at[g, 0]], xrows[b],
                              gsem[b]).wait()

    def _drain_scatter(b):
        pltpu.make_async_copy(xrows[b], accum.at[idx_all.at[0, 2]],
                              ssem[b]).wait()

    def _run_pass(xp_hbm):
        def _compute(g, b):
            xb = xrows[b]

            def _group(g2, carry):
                ty16 = idx_all[g, 1, pl.ds(g2 * LANES, LANES)]
                nv16 = nrm_all[g, pl.ds(g2 * LANES, LANES)]
                for e in range(LANES):
                    j = g2 * LANES + e
                    nv = nv16[e]
                    ty = ty16[e]
                    for v in range(VPG):
                        sl = pl.ds(v * LANES, LANES)
                        xb[j, sl] = xb[j, sl] * rel_v[ty, sl] * nv
                return carry

            lax.fori_loop(0, CK // LANES, _group, 0)

        def _outer(io, carry):
            for b in range(NBUF):
                g = io * NBUF + b
                _drain_gather(xp_hbm, g, b)
                _compute(g, b)
                pltpu.async_copy(xrows[b], accum.at[idx_all.at[g, 2]],
                                 ssem[b], add=True)
                bp = (b + 2) % NBUF

                @pl.when(jnp.logical_and(g >= 1, g + 2 < NCHUNK))
                def _pref():
                    _drain_scatter(bp)
                    _start_gather(xp_hbm, g + 2, bp)

                @pl.when(g == 0)
                def _pref0():
                    _start_gather(xp_hbm, 2, bp)

            return carry

        lax.fori_loop(0, NCHUNK // NBUF, _outer, 0)
        for b in range(NBUF):
            _drain_scatter(b)
        plsc.subcore_barrier()

    _zero_buf()
    _zero_accum()
    cpe.wait()
    cpn.wait()
    plsc.subcore_barrier()

    _start_gather(x0_hbm, 0, 0)
    _start_gather(x0_hbm, 1, 1)
    pltpu.sync_copy(rel_hbm.at[0], rel_v)
    _run_pass(x0_hbm)

    _start_gather(x1_hbm, 0, 0)
    _start_gather(x1_hbm, 1, 1)
    _writeout(0)
    _zero_buf()
    _zero_accum()
    pltpu.sync_copy(rel_hbm.at[1], rel_v)
    plsc.subcore_barrier()

    _run_pass(x1_hbm)
    _writeout(1)


def _sc_accum(x0, x1, rel_emb, edges, nrm):
    mesh = plsc.VectorSubcoreMesh(core_axis_name="c", subcore_axis_name="s")
    fn = pl.kernel(
        _sc_accum_body,
        out_type=jax.ShapeDtypeStruct((NC, 2, ENT_PAD, DH), jnp.float32),
        mesh=mesh,
        scratch_types=[
            pltpu.VMEM_SHARED((ENT_PAD, DH), jnp.float32),
            pltpu.VMEM((NREL, DH), jnp.float32),
            pltpu.VMEM((NCHUNK, 3, CK), jnp.int32),
            pltpu.VMEM((NCHUNK, CK), jnp.float32),
            pltpu.VMEM((CK, DH), jnp.float32),
            pltpu.VMEM((CK, DH), jnp.float32),
            pltpu.VMEM((CK, DH), jnp.float32),
        ] + [pltpu.SemaphoreType.DMA] * 9,
        compiler_params=pltpu.CompilerParams(use_tc_tiling_on_sc=False),
    )
    relh = rel_emb.reshape(NREL, 2, DH).transpose(1, 0, 2)
    return fn(x0, x1, relh, edges, nrm)


def _tc_layer_body(acc, x, rel, in_w, out_w, loop_w, loop_rel, bias,
                   gamma, beta, w_rel, xo, x0o, x1o, ro):
    f32 = jnp.float32
    agg = jnp.dot(acc[0, 0, :NUM_ENT, :], in_w[:DH, :],
                  preferred_element_type=f32)
    agg = agg + jnp.dot(acc[0, 1, :NUM_ENT, :], in_w[DH:, :],
                        preferred_element_type=f32)
    agg = agg + jnp.dot(acc[1, 0, :NUM_ENT, :], out_w[:DH, :],
                        preferred_element_type=f32)
    agg = agg + jnp.dot(acc[1, 1, :NUM_ENT, :], out_w[DH:, :],
                        preferred_element_type=f32)
    xl = x[...] * loop_rel[...]
    agg = agg + jnp.dot(xl, loop_w[...], preferred_element_type=f32) / 3.0
    xn = agg + bias[...]
    mu = jnp.mean(xn, axis=0, keepdims=True)
    var = jnp.mean((xn - mu) * (xn - mu), axis=0, keepdims=True)
    xn = (xn - mu) * lax.rsqrt(var + EPS) * gamma[...] + beta[...]
    xt = jnp.tanh(xn)
    xo[...] = xt
    x0o[...] = xt[:, :DH]
    x1o[...] = xt[:, DH:]
    ro[...] = jnp.dot(rel[...], w_rel[...], preferred_element_type=f32)


def _tc_layer(acc, x, rel, in_w, out_w, loop_w, loop_rel, bias, gamma,
              beta, w_rel):
    nrel = rel.shape[0]
    return pl.pallas_call(
        _tc_layer_body,
        out_shape=(
            jax.ShapeDtypeStruct((NUM_ENT, D), jnp.float32),
            jax.ShapeDtypeStruct((NUM_ENT, DH), jnp.float32),
            jax.ShapeDtypeStruct((NUM_ENT, DH), jnp.float32),
            jax.ShapeDtypeStruct((nrel, D), jnp.float32),
        ),
    )(acc, x, rel, in_w, out_w, loop_w, loop_rel.reshape(1, D),
      bias.reshape(1, D), gamma.reshape(1, D), beta.reshape(1, D), w_rel)


def _tc_relmm_body(a, b, o):
    o[...] = jnp.dot(a[...], b[...], preferred_element_type=jnp.float32)


def _tc_relmm(a, b):
    return pl.pallas_call(
        _tc_relmm_body,
        out_shape=jax.ShapeDtypeStruct((a.shape[0], b.shape[1]), jnp.float32),
    )(a, b)


BPW = BATCH // (NC * NS)


def _sc_gather_body(x_hbm, r_hbm, subj_hbm, rel_hbm, obj_hbm,
                    so_hbm, ro_hbm, oo_hbm, idx_v, rows_v, sem):
    c = lax.axis_index("c")
    s = lax.axis_index("s")
    base = (s * NC + c) * BPW
    for ih, table, oh in ((subj_hbm, x_hbm, so_hbm),
                          (rel_hbm, r_hbm, ro_hbm),
                          (obj_hbm, x_hbm, oo_hbm)):
        pltpu.sync_copy(ih.at[pl.ds(base, BPW)], idx_v)
        pltpu.async_copy(table.at[idx_v], rows_v, sem).wait()
        pltpu.sync_copy(rows_v, oh.at[pl.ds(base, BPW)])


def _sc_gather(x, r, subj, rel, obj):
    mesh = plsc.VectorSubcoreMesh(core_axis_name="c", subcore_axis_name="s")
    fn = pl.kernel(
        _sc_gather_body,
        out_type=(
            jax.ShapeDtypeStruct((BATCH, D), jnp.float32),
            jax.ShapeDtypeStruct((BATCH, D), jnp.float32),
            jax.ShapeDtypeStruct((BATCH, D), jnp.float32),
        ),
        mesh=mesh,
        scratch_types=[
            pltpu.VMEM((BPW,), jnp.int32),
            pltpu.VMEM((BPW, D), jnp.float32),
            pltpu.SemaphoreType.DMA,
        ],
    )
    return fn(x, r, subj, rel, obj)


def _prep_edges(edge_index, edge_type, edge_norm):
    pad = NS * EPT - HALF

    def half(a, fill):
        a0 = jnp.concatenate([a[:HALF], jnp.full((pad,), fill, a.dtype)])
        a1 = jnp.concatenate([a[HALF:], jnp.full((pad,), fill, a.dtype)])
        return jnp.stack([a0, a1]).reshape(NC, NS, NCHUNK, CK)

    src = half(edge_index[0].astype(jnp.int32), 0)
    typ = half(edge_type.astype(jnp.int32), 0)
    dst = half(edge_index[1].astype(jnp.int32), 0)
    nrm = half(edge_norm.astype(jnp.float32), 0.0)
    return jnp.stack([src, typ, dst], axis=3), nrm


def kernel(edge_index, edge_type, edge_norm, subj, rel, obj, init_embed,
           init_rel, rel_wt, in_w1, out_w1, loop_w1, w_rel1, loop_rel1, bias1,
           gamma1, beta1, in_w2, out_w2, loop_w2, w_rel2, loop_rel2, bias2,
           gamma2, beta2):
    edges, nrm = _prep_edges(edge_index, edge_type, edge_norm)

    rel1 = _tc_relmm(rel_wt, init_rel)

    acc = _sc_accum(init_embed[:, :DH], init_embed[:, DH:], rel1, edges, nrm)
    x1, x10, x11, rel2 = _tc_layer(acc, init_embed, rel1,
                                   in_w1, out_w1, loop_w1, loop_rel1, bias1,
                                   gamma1, beta1, w_rel1)

    acc = _sc_accum(x10, x11, rel2, edges, nrm)
    x2, _, _, rel3 = _tc_layer(acc, x1, rel2,
                               in_w2, out_w2, loop_w2, loop_rel2, bias2,
                               gamma2, beta2, w_rel2)

    return _sc_gather(x2, rel3, subj.astype(jnp.int32), rel.astype(jnp.int32),
                      obj.astype(jnp.int32))
